# Initial kernel scaffold; baseline (speedup 1.0000x reference)
import jax
import jax.numpy as jnp
from jax.experimental import pallas as pl


def kernel(x):
    raise NotImplementedError("write your pallas kernel here")


# speedup vs baseline: 1.5637x; 1.5637x over previous
import functools

import numpy as np
import jax
import jax.numpy as jnp
from jax import lax
from jax.experimental import pallas as pl
from jax.experimental.pallas import tpu as pltpu
from jax.experimental.pallas import tpu_sc as plsc

NC, NS = 2, 16
NW = NC * NS
D = 128
B, S, K = 4096, 200, 50
R = B * K
RW = R // NW
C = 128
NCH = RW // C


def _make_idx():
    r = np.arange(R, dtype=np.int64)
    idx = (r // K) * S + (r % K) * 4
    return idx.reshape(NW, NCH, C).astype(np.int32)


_IDX = _make_idx()


_mesh = plsc.VectorSubcoreMesh(core_axis_name="c", subcore_axis_name="s")


@functools.partial(
    pl.kernel,
    out_type=jax.ShapeDtypeStruct((R, D), jnp.float32),
    mesh=_mesh,
    scratch_types=[
        pltpu.VMEM((NCH, C), jnp.int32),
        pltpu.VMEM((C, D), jnp.float32),
        pltpu.SemaphoreType.DMA,
    ],
)
def _gather_sc(x_hbm, idx_hbm, out_hbm, idx_v, buf, sem):
    c = lax.axis_index("c")
    s = lax.axis_index("s")
    wid = c * NS + s
    base = wid * RW
    pltpu.sync_copy(idx_hbm.at[wid], idx_v)

    @pl.loop(0, NCH)
    def _chunk(j):
        pltpu.async_copy(x_hbm.at[idx_v.at[j]], buf, sem).wait()
        pltpu.sync_copy(buf, out_hbm.at[pl.ds(base + j * C, C)])


def kernel(x):
    x2 = x.reshape(B * S, D)
    out2 = _gather_sc(x2, _IDX)
    return out2.reshape(B, K, D)


# speedup vs baseline: 1.7625x; 1.1272x over previous
import functools

import numpy as np
import jax
import jax.numpy as jnp
from jax import lax
from jax.experimental import pallas as pl
from jax.experimental.pallas import tpu as pltpu
from jax.experimental.pallas import tpu_sc as plsc

NC, NS = 2, 16
NW = NC * NS
D = 128
B, S, K = 4096, 200, 50
R = B * K
RW = R // NW
C = 128
NCH = RW // C


def _make_idx():
    r = np.arange(R, dtype=np.int64)
    idx = (r // K) * S + (r % K) * 4
    return idx.reshape(NW, NCH, C).astype(np.int32)


_IDX = _make_idx()


NBUF = 5
NR = NCH // NBUF

_mesh = plsc.VectorSubcoreMesh(core_axis_name="c", subcore_axis_name="s")


@functools.partial(
    pl.kernel,
    out_type=jax.ShapeDtypeStruct((R, D), jnp.float32),
    mesh=_mesh,
    scratch_types=[
        pltpu.VMEM((NCH, C), jnp.int32),
        [pltpu.VMEM((C, D), jnp.float32)] * NBUF,
        [pltpu.SemaphoreType.DMA] * NBUF,
        [pltpu.SemaphoreType.DMA] * NBUF,
    ],
)
def _gather_sc(x_hbm, idx_hbm, out_hbm, idx_v, bufs, gsems, ssems):
    c = lax.axis_index("c")
    s = lax.axis_index("s")
    wid = c * NS + s
    base = wid * RW
    pltpu.sync_copy(idx_hbm.at[wid], idx_v)

    for b in range(NBUF):
        pltpu.async_copy(x_hbm.at[idx_v.at[b]], bufs[b], gsems[b])

    @pl.loop(0, NR)
    def _round(r):
        for b in range(NBUF):
            j = r * NBUF + b
            pltpu.make_async_copy(x_hbm.at[idx_v.at[j]], bufs[b], gsems[b]).wait()
            st = pltpu.async_copy(
                bufs[b], out_hbm.at[pl.ds(base + j * C, C)], ssems[b]
            )

            @pl.when(r < NR - 1)
            def _refill():
                st.wait()
                pltpu.async_copy(x_hbm.at[idx_v.at[j + NBUF]], bufs[b], gsems[b])

    for b in range(NBUF):
        j = (NR - 1) * NBUF + b
        pltpu.make_async_copy(
            bufs[b], out_hbm.at[pl.ds(base + j * C, C)], ssems[b]
        ).wait()


def kernel(x):
    x2 = x.reshape(B * S, D)
    out2 = _gather_sc(x2, _IDX)
    return out2.reshape(B, K, D)


# speedup vs baseline: 3.1091x; 1.7640x over previous
import functools

import numpy as np
import jax
import jax.numpy as jnp
from jax import lax
from jax.experimental import pallas as pl
from jax.experimental.pallas import tpu as pltpu
from jax.experimental.pallas import tpu_sc as plsc

NC, NS = 2, 16
NW = NC * NS
D = 128
B, S, K = 4096, 200, 50
BB = B // NW
NBUF = 4
NR = BB // NBUF


def _make_idx():
    b = np.arange(B, dtype=np.int64)[:, None]
    k = np.arange(K, dtype=np.int64)[None, :]
    idx = b * S + 4 * k
    return idx.reshape(NW, BB, K).astype(np.int32)


_IDX = _make_idx()

_mesh = plsc.VectorSubcoreMesh(core_axis_name="c", subcore_axis_name="s")


@functools.partial(
    pl.kernel,
    out_type=jax.ShapeDtypeStruct((B, K, D), jnp.float32),
    mesh=_mesh,
    scratch_types=[
        pltpu.VMEM((BB, K), jnp.int32),
        [pltpu.VMEM((K, D), jnp.float32)] * NBUF,
        [pltpu.SemaphoreType.DMA] * NBUF,
        [pltpu.SemaphoreType.DMA] * NBUF,
    ],
)
def _gather_sc(x_hbm, idx_hbm, out_hbm, idx_v, bufs, gsems, ssems):
    c = lax.axis_index("c")
    s = lax.axis_index("s")
    wid = c * NS + s
    base = wid * BB
    pltpu.sync_copy(idx_hbm.at[wid], idx_v)

    for b in range(NBUF):
        pltpu.async_copy(x_hbm.at[idx_v.at[b]], bufs[b], gsems[b])

    @pl.loop(0, NR)
    def _round(r):
        for b in range(NBUF):
            i = r * NBUF + b
            pltpu.make_async_copy(x_hbm.at[idx_v.at[i]], bufs[b], gsems[b]).wait()
            st = pltpu.async_copy(bufs[b], out_hbm.at[base + i], ssems[b])

            @pl.when(r < NR - 1)
            def _refill():
                st.wait()
                pltpu.async_copy(x_hbm.at[idx_v.at[i + NBUF]], bufs[b], gsems[b])

    for b in range(NBUF):
        i = (NR - 1) * NBUF + b
        pltpu.make_async_copy(bufs[b], out_hbm.at[base + i], ssems[b]).wait()


def kernel(x):
    x2 = x.reshape(B * S, D)
    return _gather_sc(x2, _IDX)
